# baseline (speedup 1.0000x reference)
import functools

import jax
import jax.numpy as jnp
from jax import lax
from jax.experimental import pallas as pl
from jax.experimental.pallas import tpu as pltpu
from jax.experimental.pallas import tpu_sc as plsc

NC, NS = 2, 16
NW = NC * NS
D = 32
B = 4096 * 50
BPW = B // NW

_mesh = plsc.VectorSubcoreMesh(core_axis_name="c", subcore_axis_name="s")


@functools.partial(
    pl.kernel,
    out_type=jax.ShapeDtypeStruct((D, B), jnp.float32),
    mesh=_mesh,
    scratch_types=[
        pltpu.VMEM((1, BPW), jnp.int32),
        pltpu.VMEM((2, BPW), jnp.float32),
        pltpu.SemaphoreType.DMA((2,)),
        pltpu.SemaphoreType.DMA((2,)),
    ],
    compiler_params=pltpu.CompilerParams(use_tc_tiling_on_sc=False),
)
def _gather_kernel(idx_hbm, table_hbm, out_hbm, idx_v, vals_v, gsem, ssem):
    wid = lax.axis_index("s") * NC + lax.axis_index("c")
    base = wid * BPW
    my_idx = idx_v.at[0]
    pltpu.sync_copy(idx_hbm.at[wid], idx_v)

    def fire_gather(c):
        pltpu.async_copy(table_hbm.at[c].at[my_idx], vals_v.at[c % 2],
                         gsem.at[c % 2])

    def fire_store(c):
        pltpu.async_copy(vals_v.at[c % 2],
                         out_hbm.at[c, pl.ds(base, BPW)],
                         ssem.at[c % 2])

    def wait_gather(c):
        pltpu.make_async_copy(table_hbm.at[c].at[my_idx], vals_v.at[c % 2],
                              gsem.at[c % 2]).wait()

    def wait_store(c):
        pltpu.make_async_copy(vals_v.at[c % 2],
                              out_hbm.at[c, pl.ds(base, BPW)],
                              ssem.at[c % 2]).wait()

    fire_gather(0)
    for c in range(D):
        if c + 1 < D:
            if c >= 1:
                wait_store(c - 1)
            fire_gather(c + 1)
        wait_gather(c)
        fire_store(c)
    wait_store(D - 2)
    wait_store(D - 1)


def kernel(inputs, table):
    idx = inputs.reshape(NW, 1, BPW).astype(jnp.int32)
    out_t = _gather_kernel(idx, table.T)
    return out_t.T


# speedup vs baseline: 15.3904x; 15.3904x over previous
import functools

import jax
import jax.numpy as jnp
from jax import lax
from jax.experimental import pallas as pl
from jax.experimental.pallas import tpu as pltpu
from jax.experimental.pallas import tpu_sc as plsc

NC, NS = 2, 16
D = 32
V = 1000000
B = 4096 * 50
BPT = B // NS
DPS = D // NC

_mesh = plsc.VectorSubcoreMesh(core_axis_name="c", subcore_axis_name="s")


@functools.partial(
    pl.kernel,
    out_type=jax.ShapeDtypeStruct((D, B), jnp.float32),
    mesh=_mesh,
    scratch_types=[
        pltpu.VMEM_SHARED((V,), jnp.float32),
        pltpu.VMEM((1, BPT), jnp.int32),
        pltpu.VMEM((BPT,), jnp.float32),
        pltpu.SemaphoreType.DMA,
    ],
)
def _gather_kernel(idx_hbm, table_hbm, out_hbm, slab, idx_v, vals_v, sem):
    sc = lax.axis_index("c")
    sid = lax.axis_index("s")
    pltpu.sync_copy(idx_hbm.at[sid], idx_v)

    for k in range(DPS):
        c = sc * DPS + k

        @pl.when(sid == 0)
        def _stage():
            pltpu.sync_copy(table_hbm.at[c], slab)

        plsc.subcore_barrier()
        pltpu.async_copy(slab.at[idx_v.at[0]], vals_v, sem).wait()
        pltpu.sync_copy(vals_v, out_hbm.at[c, pl.ds(sid * BPT, BPT)])
        plsc.subcore_barrier()


def kernel(inputs, table):
    idx = inputs.reshape(NS, 1, BPT).astype(jnp.int32)
    out_t = _gather_kernel(idx, table.T)
    return out_t.T


# speedup vs baseline: 15.7919x; 1.0261x over previous
import functools

import jax
import jax.numpy as jnp
from jax import lax
from jax.experimental import pallas as pl
from jax.experimental.pallas import tpu as pltpu
from jax.experimental.pallas import tpu_sc as plsc

NC, NS = 2, 16
D = 32
V = 1000000
B = 4096 * 50
BPT = B // NS
DPS = D // NC
CH = 62464
LCH = 62976
TAIL = 128

_mesh = plsc.VectorSubcoreMesh(core_axis_name="c", subcore_axis_name="s")


@functools.partial(
    pl.kernel,
    out_type=jax.ShapeDtypeStruct((D, B), jnp.float32),
    mesh=_mesh,
    scratch_types=[
        pltpu.VMEM_SHARED((V,), jnp.float32),
        pltpu.VMEM((1, BPT), jnp.int32),
        pltpu.VMEM((BPT,), jnp.float32),
        pltpu.VMEM((BPT,), jnp.float32),
        pltpu.VMEM((TAIL,), jnp.float32),
        pltpu.SemaphoreType.DMA,
        pltpu.SemaphoreType.DMA,
        pltpu.SemaphoreType.DMA((2,)),
    ],
)
def _gather_kernel(idx_hbm, table_hbm, tail_hbm, out_hbm, slab, idx_v,
                   vals0, vals1, tail_v, ssem, gsem, osem):
    sc = lax.axis_index("c")
    sid = lax.axis_index("s")
    vals = (vals0, vals1)
    pltpu.sync_copy(idx_hbm.at[sid], idx_v)

    def plane(k):
        return sc * DPS + k

    def stage_start(k):
        @pl.when(sid < NS - 1)
        def _mid():
            pltpu.async_copy(table_hbm.at[plane(k), pl.ds(sid * CH, CH)],
                             slab.at[pl.ds(sid * CH, CH)], ssem)

        @pl.when(sid == NS - 1)
        def _last():
            pltpu.async_copy(
                table_hbm.at[plane(k), pl.ds((NS - 1) * CH, LCH)],
                slab.at[pl.ds((NS - 1) * CH, LCH)], ssem)
            pltpu.async_copy(tail_hbm.at[pl.ds(plane(k) * TAIL, TAIL)],
                             tail_v, ssem)

    def stage_wait(k):
        @pl.when(sid < NS - 1)
        def _mid():
            pltpu.make_async_copy(
                table_hbm.at[plane(k), pl.ds(sid * CH, CH)],
                slab.at[pl.ds(sid * CH, CH)], ssem).wait()

        @pl.when(sid == NS - 1)
        def _last():
            pltpu.make_async_copy(
                table_hbm.at[plane(k), pl.ds((NS - 1) * CH, LCH)],
                slab.at[pl.ds((NS - 1) * CH, LCH)], ssem).wait()
            pltpu.make_async_copy(tail_hbm.at[pl.ds(plane(k) * TAIL, TAIL)],
                                  tail_v, ssem).wait()
            pltpu.sync_copy(tail_v, slab.at[pl.ds(V - TAIL, TAIL)])

    def out_ref(k):
        return out_hbm.at[plane(k), pl.ds(sid * BPT, BPT)]

    stage_start(0)
    for k in range(DPS):
        b = k % 2
        stage_wait(k)
        plsc.subcore_barrier()
        if k >= 2:
            pltpu.make_async_copy(vals[b], out_ref(k - 2), osem.at[b]).wait()
        pltpu.async_copy(slab.at[idx_v.at[0]], vals[b], gsem).wait()
        pltpu.async_copy(vals[b], out_ref(k), osem.at[b])
        plsc.subcore_barrier()
        if k + 1 < DPS:
            stage_start(k + 1)
    pltpu.make_async_copy(vals[DPS % 2], out_ref(DPS - 2),
                          osem.at[DPS % 2]).wait()
    pltpu.make_async_copy(vals[(DPS - 1) % 2], out_ref(DPS - 1),
                          osem.at[(DPS - 1) % 2]).wait()


def kernel(inputs, table):
    idx = inputs.reshape(NS, 1, BPT).astype(jnp.int32)
    table_t = table.T
    tail = table_t[:, V - TAIL:].reshape(D * TAIL)
    out_t = _gather_kernel(idx, table_t, tail)
    return out_t.T
